# baseline (speedup 1.0000x reference)
import jax
import jax.numpy as jnp
from jax import lax
from jax.experimental import pallas as pl
from jax.experimental.pallas import tpu as pltpu
from jax.experimental.pallas import tpu_sc as plsc

N = 10000
E = 320000
D = 128
H = 64
G = 64
C = 10

NS = 16
L = 16

NPAD = 10240
NSL = NPAD // NS
CNT_OFF = G * N
ASZ = G * N + 128
SL = ASZ // NS
SL2 = 8008
EC = E // NS
CB = 1024
RB = CB // 128
NB = (EC + CB - 1) // CB
ZB = 4016


def _invsqrt(v):
    i = lax.bitcast_convert_type(v, jnp.int32)
    i = jnp.int32(0x5F3759DF) - (i >> 1)
    y = lax.bitcast_convert_type(i, jnp.float32)
    for _ in range(3):
        y = y * (1.5 - 0.5 * v * y * y)
    return y


def _sc_body(src_hbm, dst_hbm, batch_hbm, out_hbm,
             batch_v, dinv_v, edge_v, degsl_v, idx0, val0, idx1, val1,
             idx2, val2, idx3, val3, onesf_v, onest_v,
             zeros_v, stage_v, sem0, sem1, sem2, sem3,
             a_sp, deg_sp, dinv_sp):
    s = lax.axis_index("s")
    iota = lax.iota(jnp.int32, L)
    zero16 = jnp.zeros((L,), jnp.float32)
    idxs = (idx0, idx1, idx2, idx3)
    vals = (val0, val1, val2, val3)
    sems = (sem0, sem1, sem2, sem3)

    pltpu.sync_copy(batch_hbm, batch_v)
    pltpu.sync_copy(src_hbm.at[pl.ds(s * EC, EC)], edge_v.at[pl.ds(0, EC)])
    pltpu.sync_copy(dst_hbm.at[pl.ds(s * EC, EC)], edge_v.at[pl.ds(EC, EC)])

    def zloop(i, _):
        zeros_v[pl.ds(i * L, L)] = zero16
        return 0
    lax.fori_loop(0, ZB // L, zloop, 0)

    tail_valid = EC - (NB - 1) * CB

    def oloop(i, _):
        onesf_v[pl.ds(i * L, L)] = jnp.full((L,), 1.0, jnp.float32)
        onest_v[pl.ds(i * L, L)] = jnp.where(i * L + iota < tail_valid,
                                             1.0, 0.0)
        return 0
    lax.fori_loop(0, CB // L, oloop, 0)

    base = s * SL
    for j in range(9):
        pltpu.sync_copy(zeros_v.at[pl.ds(0, 4000)],
                        a_sp.at[pl.ds(base + j * 4000, 4000)])
    pltpu.sync_copy(zeros_v.at[pl.ds(0, SL - 36000)],
                    a_sp.at[pl.ds(base + 36000, SL - 36000)])
    pltpu.sync_copy(zeros_v.at[pl.ds(0, NSL)], deg_sp.at[pl.ds(s * NSL, NSL)])
    plsc.subcore_barrier()

    def fill_deg(idx_b, b):
        for k in range(CB // L):
            e0c = jnp.minimum(b * CB + k * L, EC - L)
            idx_b[pl.ds(k * L, L)] = edge_v[pl.ds(EC + e0c, L)]

    def fill(idx_b, val_b, b):
        for k in range(CB // L):
            e0 = b * CB + k * L
            e0c = jnp.minimum(e0, EC - L)
            ok = (e0 + iota) < EC
            col = k * L
            d16 = edge_v[pl.ds(EC + e0c, L)]
            s16 = edge_v[pl.ds(e0c, L)]
            dvs = plsc.load_gather(dinv_v, [s16])
            dvd = plsc.load_gather(dinv_v, [d16])
            g16 = plsc.load_gather(batch_v, [d16])
            idx_b[pl.ds(col, L)] = g16 * N + s16
            val_b[pl.ds(col, L)] = jnp.where(ok, dvs * dvd, 0.0)

    def p1(i, _):
        descs = []
        for q in range(4):
            fill_deg(idxs[q], 4 * i + q)
            descs.append(pltpu.async_copy(onesf_v, deg_sp.at[idxs[q]],
                                          sems[q], add=True))
        for d in descs:
            d.wait()
        return 0
    lax.fori_loop(0, NB // 4 - 1, p1, 0)
    descs = []
    for q in range(4):
        fill_deg(idxs[q], NB - 4 + q)
        vref = onest_v if q == 3 else onesf_v
        descs.append(pltpu.async_copy(vref, deg_sp.at[idxs[q]],
                                      sems[q], add=True))
    for d in descs:
        d.wait()
    plsc.subcore_barrier()

    pltpu.sync_copy(deg_sp.at[pl.ds(s * NSL, NSL)], degsl_v)

    def dloop(jj, _):
        dg = degsl_v[pl.ds(jj * L, L)] + 1.0
        dinv_v[pl.ds(s * NSL + jj * L, L)] = _invsqrt(dg)
        return 0
    lax.fori_loop(0, NSL // L, dloop, 0)
    pltpu.sync_copy(dinv_v.at[pl.ds(s * NSL, NSL)],
                    dinv_sp.at[pl.ds(s * NSL, NSL)])
    plsc.subcore_barrier()
    pltpu.sync_copy(dinv_sp, dinv_v)

    def p2(i, _):
        descs = []
        for q in range(4):
            fill(idxs[q], vals[q], 4 * i + q)
            descs.append(pltpu.async_copy(vals[q], a_sp.at[idxs[q]],
                                          sems[q], add=True))
        for d in descs:
            d.wait()
        return 0
    lax.fori_loop(0, NB // 4, p2, 0)

    descs = []
    for half, (idx_b, val_b, sm) in enumerate(
            ((idx0, val0, sem0), (idx1, val1, sem1))):
        for jj in range(20):
            j = s * 40 + half * 20 + jj
            jc = jnp.minimum(j, N // L - 1)
            ok = (j * L + iota) < N
            i16 = jc * L + iota
            g16 = batch_v[pl.ds(jc * L, L)]
            dv = dinv_v[pl.ds(jc * L, L)]
            m, m2 = 2 * jj, 2 * jj + 1
            idx_b[pl.ds(m * L, L)] = g16 * N + i16
            val_b[pl.ds(m * L, L)] = jnp.where(ok, dv * dv, 0.0)
            idx_b[pl.ds(m2 * L, L)] = CNT_OFF + g16
            val_b[pl.ds(m2 * L, L)] = jnp.where(ok, 1.0, 0.0)
        for m in range(40, CB // L):
            val_b[pl.ds(m * L, L)] = zero16
        descs.append(pltpu.async_copy(val_b, a_sp.at[idx_b], sm, add=True))
    for d in descs:
        d.wait()
    plsc.subcore_barrier()

    pltpu.sync_copy(a_sp.at[pl.ds(s * SL, SL2)], stage_v)
    pltpu.sync_copy(stage_v, out_hbm.at[pl.ds(s * SL, SL2)])
    for j in range(4):
        off = s * SL + SL2 + j * 8000
        pltpu.sync_copy(a_sp.at[pl.ds(off, 8000)], stage_v.at[pl.ds(0, 8000)])
        pltpu.sync_copy(stage_v.at[pl.ds(0, 8000)], out_hbm.at[pl.ds(off, 8000)])


def _h_body(x_ref, w1_ref, o_ref):
    o_ref[...] = jnp.dot(x_ref[...], w1_ref[...],
                         preferred_element_type=jnp.float32)


def _tc_body(ng_ref, a_ref, cnt_ref, h_ref, b1_ref, w2_ref, b2_ref,
             o_ref):
    z = jnp.dot(a_ref[...], h_ref[...], preferred_element_type=jnp.float32)
    cnt = cnt_ref[...]
    sums = z + cnt * b1_ref[...]
    valid = lax.broadcasted_iota(jnp.int32, (G, 1), 0) < ng_ref[0, 0]
    sums = jnp.where(valid, sums, 0.0)
    cntv = jnp.where(valid, cnt, 0.0)
    pooled = sums / jnp.maximum(cntv, 1.0)
    logits = jnp.dot(pooled, w2_ref[...],
                     preferred_element_type=jnp.float32) + b2_ref[...]
    mx = jnp.max(logits, axis=1, keepdims=True)
    lse = mx + jnp.log(jnp.sum(jnp.exp(logits - mx), axis=1, keepdims=True))
    o_ref[...] = logits - lse


def kernel(x, edge_index, batch, num_graphs, W1, b1, W2, b2):
    mesh = plsc.VectorSubcoreMesh(core_axis_name="c", subcore_axis_name="s",
                                  num_cores=1)
    sc = pl.kernel(
        _sc_body,
        out_type=jax.ShapeDtypeStruct((ASZ,), jnp.float32),
        mesh=mesh,
        compiler_params=pltpu.CompilerParams(needs_layout_passes=False),
        scratch_types=[
            pltpu.VMEM((N,), jnp.int32),
            pltpu.VMEM((NPAD,), jnp.float32),
            pltpu.VMEM((2 * EC,), jnp.int32),
            pltpu.VMEM((NSL,), jnp.float32),
            pltpu.VMEM((CB,), jnp.int32),
            pltpu.VMEM((CB,), jnp.float32),
            pltpu.VMEM((CB,), jnp.int32),
            pltpu.VMEM((CB,), jnp.float32),
            pltpu.VMEM((CB,), jnp.int32),
            pltpu.VMEM((CB,), jnp.float32),
            pltpu.VMEM((CB,), jnp.int32),
            pltpu.VMEM((CB,), jnp.float32),
            pltpu.VMEM((CB,), jnp.float32),
            pltpu.VMEM((CB,), jnp.float32),
            pltpu.VMEM((ZB,), jnp.float32),
            pltpu.VMEM((SL2,), jnp.float32),
            pltpu.SemaphoreType.DMA,
            pltpu.SemaphoreType.DMA,
            pltpu.SemaphoreType.DMA,
            pltpu.SemaphoreType.DMA,
            pltpu.VMEM_SHARED((ASZ,), jnp.float32),
            pltpu.VMEM_SHARED((NPAD,), jnp.float32),
            pltpu.VMEM_SHARED((NPAD,), jnp.float32),
        ],
    )
    h = pl.pallas_call(
        _h_body,
        out_shape=jax.ShapeDtypeStruct((N, H), jnp.float32),
    )(x, W1)
    a2 = sc(edge_index[0], edge_index[1], batch)
    amat = a2[:G * N].reshape(G, N)
    cntp = a2[CNT_OFF:CNT_OFF + G].reshape(G, 1)
    ng = jnp.asarray(num_graphs, jnp.int32).reshape(1, 1)
    return pl.pallas_call(
        _tc_body,
        out_shape=jax.ShapeDtypeStruct((G, C), jnp.float32),
    )(ng, amat, cntp, h, b1.reshape(1, H), W2, b2.reshape(1, C))


# speedup vs baseline: 1.0728x; 1.0728x over previous
import jax
import jax.numpy as jnp
from jax import lax
from jax.experimental import pallas as pl
from jax.experimental.pallas import tpu as pltpu
from jax.experimental.pallas import tpu_sc as plsc

N = 10000
E = 320000
D = 128
H = 64
G = 64
C = 10

NS = 16
L = 16

NPAD = 10240
NSL = NPAD // NS
CNT_OFF = G * N
ASZ = G * N + 128
SL = ASZ // NS
SL2 = 8008
EC = E // NS
CB = 1024
RB = CB // 128
NB = (EC + CB - 1) // CB
ZB = 4016


def _invsqrt(v):
    i = lax.bitcast_convert_type(v, jnp.int32)
    i = jnp.int32(0x5F3759DF) - (i >> 1)
    y = lax.bitcast_convert_type(i, jnp.float32)
    for _ in range(3):
        y = y * (1.5 - 0.5 * v * y * y)
    return y


def _sc_body(src_hbm, dst_hbm, batch_hbm, out_hbm,
             batch_v, dinv_v, edge_v, degsl_v, idx0, val0, idx1, val1,
             idx2, val2, idx3, val3, onesf_v, onest_v,
             zeros_v, stage_v, sem0, sem1, sem2, sem3,
             a_sp, deg_sp, dinv_sp):
    s = lax.axis_index("s")
    iota = lax.iota(jnp.int32, L)
    zero16 = jnp.zeros((L,), jnp.float32)
    idxs = (idx0, idx1, idx2, idx3)
    vals = (val0, val1, val2, val3)
    sems = (sem0, sem1, sem2, sem3)

    din = [pltpu.async_copy(batch_hbm, batch_v, sem0),
           pltpu.async_copy(src_hbm.at[pl.ds(s * EC, EC)],
                            edge_v.at[pl.ds(0, EC)], sem1),
           pltpu.async_copy(dst_hbm.at[pl.ds(s * EC, EC)],
                            edge_v.at[pl.ds(EC, EC)], sem2)]

    def zloop(i, _):
        zeros_v[pl.ds(i * L, L)] = zero16
        return 0
    lax.fori_loop(0, ZB // L, zloop, 0)

    tail_valid = EC - (NB - 1) * CB

    def oloop(i, _):
        onesf_v[pl.ds(i * L, L)] = jnp.full((L,), 1.0, jnp.float32)
        onest_v[pl.ds(i * L, L)] = jnp.where(i * L + iota < tail_valid,
                                             1.0, 0.0)
        return 0
    lax.fori_loop(0, CB // L, oloop, 0)

    base = s * SL
    zd = [pltpu.async_copy(zeros_v.at[pl.ds(0, 4000)],
                           a_sp.at[pl.ds(base + j * 4000, 4000)], sem3)
          for j in range(9)]
    zd.append(pltpu.async_copy(zeros_v.at[pl.ds(0, SL - 36000)],
                               a_sp.at[pl.ds(base + 36000, SL - 36000)], sem3))
    zd.append(pltpu.async_copy(zeros_v.at[pl.ds(0, NSL)],
                               deg_sp.at[pl.ds(s * NSL, NSL)], sem3))
    for d in din + zd:
        d.wait()
    plsc.subcore_barrier()

    def fill_deg(idx_b, b):
        for k in range(CB // L):
            e0c = jnp.minimum(b * CB + k * L, EC - L)
            idx_b[pl.ds(k * L, L)] = edge_v[pl.ds(EC + e0c, L)]

    def fill(idx_b, val_b, b):
        for k in range(CB // L):
            e0 = b * CB + k * L
            e0c = jnp.minimum(e0, EC - L)
            ok = (e0 + iota) < EC
            col = k * L
            d16 = edge_v[pl.ds(EC + e0c, L)]
            s16 = edge_v[pl.ds(e0c, L)]
            dvs = plsc.load_gather(dinv_v, [s16])
            dvd = plsc.load_gather(dinv_v, [d16])
            g16 = plsc.load_gather(batch_v, [d16])
            idx_b[pl.ds(col, L)] = g16 * N + s16
            val_b[pl.ds(col, L)] = jnp.where(ok, dvs * dvd, 0.0)

    def p1(i, _):
        descs = []
        for q in range(4):
            fill_deg(idxs[q], 4 * i + q)
            descs.append(pltpu.async_copy(onesf_v, deg_sp.at[idxs[q]],
                                          sems[q], add=True))
        for d in descs:
            d.wait()
        return 0
    lax.fori_loop(0, NB // 4 - 1, p1, 0)
    descs = []
    for q in range(4):
        fill_deg(idxs[q], NB - 4 + q)
        vref = onest_v if q == 3 else onesf_v
        descs.append(pltpu.async_copy(vref, deg_sp.at[idxs[q]],
                                      sems[q], add=True))
    for d in descs:
        d.wait()
    plsc.subcore_barrier()

    pltpu.sync_copy(deg_sp.at[pl.ds(s * NSL, NSL)], degsl_v)

    def dloop(jj, _):
        dg = degsl_v[pl.ds(jj * L, L)] + 1.0
        dinv_v[pl.ds(s * NSL + jj * L, L)] = _invsqrt(dg)
        return 0
    lax.fori_loop(0, NSL // L, dloop, 0)
    pltpu.sync_copy(dinv_v.at[pl.ds(s * NSL, NSL)],
                    dinv_sp.at[pl.ds(s * NSL, NSL)])
    plsc.subcore_barrier()
    pltpu.sync_copy(dinv_sp, dinv_v)

    def p2(i, _):
        descs = []
        for q in range(4):
            fill(idxs[q], vals[q], 4 * i + q)
            descs.append(pltpu.async_copy(vals[q], a_sp.at[idxs[q]],
                                          sems[q], add=True))
        for d in descs:
            d.wait()
        return 0
    lax.fori_loop(0, NB // 4, p2, 0)

    descs = []
    for half, (idx_b, val_b, sm) in enumerate(
            ((idx0, val0, sem0), (idx1, val1, sem1))):
        for jj in range(20):
            j = s * 40 + half * 20 + jj
            jc = jnp.minimum(j, N // L - 1)
            ok = (j * L + iota) < N
            i16 = jc * L + iota
            g16 = batch_v[pl.ds(jc * L, L)]
            dv = dinv_v[pl.ds(jc * L, L)]
            m, m2 = 2 * jj, 2 * jj + 1
            idx_b[pl.ds(m * L, L)] = g16 * N + i16
            val_b[pl.ds(m * L, L)] = jnp.where(ok, dv * dv, 0.0)
            idx_b[pl.ds(m2 * L, L)] = CNT_OFF + g16
            val_b[pl.ds(m2 * L, L)] = jnp.where(ok, 1.0, 0.0)
        for m in range(40, CB // L):
            val_b[pl.ds(m * L, L)] = zero16
        descs.append(pltpu.async_copy(val_b, a_sp.at[idx_b], sm, add=True))
    for d in descs:
        d.wait()
    plsc.subcore_barrier()

    pltpu.sync_copy(a_sp.at[pl.ds(s * SL, SL2)], stage_v)
    pltpu.sync_copy(stage_v, out_hbm.at[pl.ds(s * SL, SL2)])
    for j in range(4):
        off = s * SL + SL2 + j * 8000
        pltpu.sync_copy(a_sp.at[pl.ds(off, 8000)], stage_v.at[pl.ds(0, 8000)])
        pltpu.sync_copy(stage_v.at[pl.ds(0, 8000)], out_hbm.at[pl.ds(off, 8000)])


def _tc_body(ng_ref, a_ref, cnt_ref, x_ref, w1_ref, b1_ref, w2_ref, b2_ref,
             o_ref):
    p = jnp.dot(a_ref[...], x_ref[...], preferred_element_type=jnp.float32)
    cnt = cnt_ref[...]
    z = jnp.dot(p, w1_ref[...], preferred_element_type=jnp.float32)
    sums = z + cnt * b1_ref[...]
    valid = lax.broadcasted_iota(jnp.int32, (G, 1), 0) < ng_ref[0, 0]
    sums = jnp.where(valid, sums, 0.0)
    cntv = jnp.where(valid, cnt, 0.0)
    pooled = sums / jnp.maximum(cntv, 1.0)
    logits = jnp.dot(pooled, w2_ref[...],
                     preferred_element_type=jnp.float32) + b2_ref[...]
    mx = jnp.max(logits, axis=1, keepdims=True)
    lse = mx + jnp.log(jnp.sum(jnp.exp(logits - mx), axis=1, keepdims=True))
    o_ref[...] = logits - lse


def kernel(x, edge_index, batch, num_graphs, W1, b1, W2, b2):
    mesh = plsc.VectorSubcoreMesh(core_axis_name="c", subcore_axis_name="s",
                                  num_cores=1)
    sc = pl.kernel(
        _sc_body,
        out_type=jax.ShapeDtypeStruct((ASZ,), jnp.float32),
        mesh=mesh,
        compiler_params=pltpu.CompilerParams(needs_layout_passes=False),
        scratch_types=[
            pltpu.VMEM((N,), jnp.int32),
            pltpu.VMEM((NPAD,), jnp.float32),
            pltpu.VMEM((2 * EC,), jnp.int32),
            pltpu.VMEM((NSL,), jnp.float32),
            pltpu.VMEM((CB,), jnp.int32),
            pltpu.VMEM((CB,), jnp.float32),
            pltpu.VMEM((CB,), jnp.int32),
            pltpu.VMEM((CB,), jnp.float32),
            pltpu.VMEM((CB,), jnp.int32),
            pltpu.VMEM((CB,), jnp.float32),
            pltpu.VMEM((CB,), jnp.int32),
            pltpu.VMEM((CB,), jnp.float32),
            pltpu.VMEM((CB,), jnp.float32),
            pltpu.VMEM((CB,), jnp.float32),
            pltpu.VMEM((ZB,), jnp.float32),
            pltpu.VMEM((SL2,), jnp.float32),
            pltpu.SemaphoreType.DMA,
            pltpu.SemaphoreType.DMA,
            pltpu.SemaphoreType.DMA,
            pltpu.SemaphoreType.DMA,
            pltpu.VMEM_SHARED((ASZ,), jnp.float32),
            pltpu.VMEM_SHARED((NPAD,), jnp.float32),
            pltpu.VMEM_SHARED((NPAD,), jnp.float32),
        ],
    )
    a2 = sc(edge_index[0], edge_index[1], batch)
    amat = a2[:G * N].reshape(G, N)
    cntp = a2[CNT_OFF:CNT_OFF + G].reshape(G, 1)
    ng = jnp.asarray(num_graphs, jnp.int32).reshape(1, 1)
    return pl.pallas_call(
        _tc_body,
        out_shape=jax.ShapeDtypeStruct((G, C), jnp.float32),
    )(ng, amat, cntp, x, W1, b1.reshape(1, H), W2, b2.reshape(1, C))


# speedup vs baseline: 1.0826x; 1.0092x over previous
import jax
import jax.numpy as jnp
from jax import lax
from jax.experimental import pallas as pl
from jax.experimental.pallas import tpu as pltpu
from jax.experimental.pallas import tpu_sc as plsc

N = 10000
E = 320000
D = 128
H = 64
G = 64
C = 10

NS = 16
L = 16

NPAD = 10240
NSL = NPAD // NS
CNT_OFF = G * N
ASZ = G * N + 128
SL = ASZ // NS
SL2 = 8008
EC = E // NS
CB = 1024
RB = CB // 128
NB = (EC + CB - 1) // CB
ZB = 4016


def _invsqrt(v):
    i = lax.bitcast_convert_type(v, jnp.int32)
    i = jnp.int32(0x5F3759DF) - (i >> 1)
    y = lax.bitcast_convert_type(i, jnp.float32)
    for _ in range(3):
        y = y * (1.5 - 0.5 * v * y * y)
    return y


def _sc_body(src_hbm, dst_hbm, batch_hbm, out_hbm,
             batch_v, dinv_v, edge_v, degsl_v, idx0, val0, idx1, val1,
             idx2, val2, idx3, val3, onesf_v, onest_v,
             zeros_v, stage_v, sem0, sem1, sem2, sem3,
             a_sp, deg_sp, dinv_sp):
    s = lax.axis_index("s")
    iota = lax.iota(jnp.int32, L)
    zero16 = jnp.zeros((L,), jnp.float32)
    idxs = (idx0, idx1, idx2, idx3)
    vals = (val0, val1, val2, val3)
    sems = (sem0, sem1, sem2, sem3)

    din = [pltpu.async_copy(batch_hbm, batch_v, sem0),
           pltpu.async_copy(src_hbm.at[pl.ds(s * EC, EC)],
                            edge_v.at[pl.ds(0, EC)], sem1),
           pltpu.async_copy(dst_hbm.at[pl.ds(s * EC, EC)],
                            edge_v.at[pl.ds(EC, EC)], sem2)]

    def zloop(i, _):
        zeros_v[pl.ds(i * L, L)] = zero16
        return 0
    lax.fori_loop(0, ZB // L, zloop, 0)

    tail_valid = EC - (NB - 1) * CB

    def oloop(i, _):
        onesf_v[pl.ds(i * L, L)] = jnp.full((L,), 1.0, jnp.float32)
        onest_v[pl.ds(i * L, L)] = jnp.where(i * L + iota < tail_valid,
                                             1.0, 0.0)
        return 0
    lax.fori_loop(0, CB // L, oloop, 0)

    base = s * SL
    zd = [pltpu.async_copy(zeros_v.at[pl.ds(0, 4000)],
                           a_sp.at[pl.ds(base + j * 4000, 4000)], sem3)
          for j in range(9)]
    zd.append(pltpu.async_copy(zeros_v.at[pl.ds(0, SL - 36000)],
                               a_sp.at[pl.ds(base + 36000, SL - 36000)], sem3))
    zd.append(pltpu.async_copy(zeros_v.at[pl.ds(0, NSL)],
                               deg_sp.at[pl.ds(s * NSL, NSL)], sem3))
    for d in din + zd:
        d.wait()
    plsc.subcore_barrier()

    def fill_deg(idx_b, b):
        for k in range(CB // L):
            e0c = jnp.minimum(b * CB + k * L, EC - L)
            idx_b[pl.ds(k * L, L)] = edge_v[pl.ds(EC + e0c, L)]

    def fill(idx_b, val_b, b):
        for k in range(CB // L):
            e0 = b * CB + k * L
            e0c = jnp.minimum(e0, EC - L)
            ok = (e0 + iota) < EC
            col = k * L
            d16 = edge_v[pl.ds(EC + e0c, L)]
            s16 = edge_v[pl.ds(e0c, L)]
            dvs = plsc.load_gather(dinv_v, [s16])
            dvd = plsc.load_gather(dinv_v, [d16])
            g16 = plsc.load_gather(batch_v, [d16])
            idx_b[pl.ds(col, L)] = g16 * N + s16
            val_b[pl.ds(col, L)] = jnp.where(ok, dvs * dvd, 0.0)

    def p1(i, _):
        descs = []
        for q in range(4):
            fill_deg(idxs[q], 4 * i + q)
            descs.append(pltpu.async_copy(onesf_v, deg_sp.at[idxs[q]],
                                          sems[q], add=True))
        for d in descs:
            d.wait()
        return 0
    lax.fori_loop(0, NB // 4 - 1, p1, 0)
    descs = []
    for q in range(4):
        fill_deg(idxs[q], NB - 4 + q)
        vref = onest_v if q == 3 else onesf_v
        descs.append(pltpu.async_copy(vref, deg_sp.at[idxs[q]],
                                      sems[q], add=True))
    for d in descs:
        d.wait()
    plsc.subcore_barrier()

    pltpu.sync_copy(deg_sp.at[pl.ds(s * NSL, NSL)], degsl_v)

    def dloop(jj, _):
        dg = degsl_v[pl.ds(jj * L, L)] + 1.0
        dinv_v[pl.ds(s * NSL + jj * L, L)] = _invsqrt(dg)
        return 0
    lax.fori_loop(0, NSL // L, dloop, 0)
    pltpu.sync_copy(dinv_v.at[pl.ds(s * NSL, NSL)],
                    dinv_sp.at[pl.ds(s * NSL, NSL)])
    plsc.subcore_barrier()
    pltpu.sync_copy(dinv_sp, dinv_v)

    def p2(i, _):
        descs = []
        for q in range(4):
            fill(idxs[q], vals[q], 4 * i + q)
            descs.append(pltpu.async_copy(vals[q], a_sp.at[idxs[q]],
                                          sems[q], add=True))
        for d in descs:
            d.wait()
        return 0
    lax.fori_loop(0, NB // 4, p2, 0)

    descs = []
    for half, (idx_b, val_b, sm) in enumerate(
            ((idx0, val0, sem0), (idx1, val1, sem1))):
        for jj in range(20):
            j = s * 40 + half * 20 + jj
            jc = jnp.minimum(j, N // L - 1)
            ok = (j * L + iota) < N
            i16 = jc * L + iota
            g16 = batch_v[pl.ds(jc * L, L)]
            dv = dinv_v[pl.ds(jc * L, L)]
            m, m2 = 2 * jj, 2 * jj + 1
            idx_b[pl.ds(m * L, L)] = g16 * N + i16
            val_b[pl.ds(m * L, L)] = jnp.where(ok, dv * dv, 0.0)
            idx_b[pl.ds(m2 * L, L)] = CNT_OFF + g16
            val_b[pl.ds(m2 * L, L)] = jnp.where(ok, 1.0, 0.0)
        for m in range(40, CB // L):
            val_b[pl.ds(m * L, L)] = zero16
        descs.append(pltpu.async_copy(val_b, a_sp.at[idx_b], sm, add=True))
    for d in descs:
        d.wait()
    plsc.subcore_barrier()

    outs = {}
    for j in range(11):
        buf = stage_v if j % 2 == 0 else zeros_v
        sz = 4000 if j < 10 else 8
        off = s * SL + j * 4000
        if j >= 2:
            outs[j - 2].wait()
        pltpu.async_copy(a_sp.at[pl.ds(off, sz)], buf.at[pl.ds(0, sz)],
                         sem0).wait()
        outs[j] = pltpu.async_copy(buf.at[pl.ds(0, sz)],
                                   out_hbm.at[pl.ds(off, sz)], sem1)
    outs[9].wait()
    outs[10].wait()


def _tc_body(ng_ref, a_ref, cnt_ref, x_ref, w1_ref, b1_ref, w2_ref, b2_ref,
             o_ref):
    p = jnp.dot(a_ref[...], x_ref[...], preferred_element_type=jnp.float32)
    cnt = cnt_ref[...]
    z = jnp.dot(p, w1_ref[...], preferred_element_type=jnp.float32)
    sums = z + cnt * b1_ref[...]
    valid = lax.broadcasted_iota(jnp.int32, (G, 1), 0) < ng_ref[0, 0]
    sums = jnp.where(valid, sums, 0.0)
    cntv = jnp.where(valid, cnt, 0.0)
    pooled = sums / jnp.maximum(cntv, 1.0)
    logits = jnp.dot(pooled, w2_ref[...],
                     preferred_element_type=jnp.float32) + b2_ref[...]
    mx = jnp.max(logits, axis=1, keepdims=True)
    lse = mx + jnp.log(jnp.sum(jnp.exp(logits - mx), axis=1, keepdims=True))
    o_ref[...] = logits - lse


def kernel(x, edge_index, batch, num_graphs, W1, b1, W2, b2):
    mesh = plsc.VectorSubcoreMesh(core_axis_name="c", subcore_axis_name="s",
                                  num_cores=1)
    sc = pl.kernel(
        _sc_body,
        out_type=jax.ShapeDtypeStruct((ASZ,), jnp.float32),
        mesh=mesh,
        compiler_params=pltpu.CompilerParams(needs_layout_passes=False),
        scratch_types=[
            pltpu.VMEM((N,), jnp.int32),
            pltpu.VMEM((NPAD,), jnp.float32),
            pltpu.VMEM((2 * EC,), jnp.int32),
            pltpu.VMEM((NSL,), jnp.float32),
            pltpu.VMEM((CB,), jnp.int32),
            pltpu.VMEM((CB,), jnp.float32),
            pltpu.VMEM((CB,), jnp.int32),
            pltpu.VMEM((CB,), jnp.float32),
            pltpu.VMEM((CB,), jnp.int32),
            pltpu.VMEM((CB,), jnp.float32),
            pltpu.VMEM((CB,), jnp.int32),
            pltpu.VMEM((CB,), jnp.float32),
            pltpu.VMEM((CB,), jnp.float32),
            pltpu.VMEM((CB,), jnp.float32),
            pltpu.VMEM((ZB,), jnp.float32),
            pltpu.VMEM((SL2,), jnp.float32),
            pltpu.SemaphoreType.DMA,
            pltpu.SemaphoreType.DMA,
            pltpu.SemaphoreType.DMA,
            pltpu.SemaphoreType.DMA,
            pltpu.VMEM_SHARED((ASZ,), jnp.float32),
            pltpu.VMEM_SHARED((NPAD,), jnp.float32),
            pltpu.VMEM_SHARED((NPAD,), jnp.float32),
        ],
    )
    a2 = sc(edge_index[0], edge_index[1], batch)
    amat = a2[:G * N].reshape(G, N)
    cntp = a2[CNT_OFF:CNT_OFF + G].reshape(G, 1)
    ng = jnp.asarray(num_graphs, jnp.int32).reshape(1, 1)
    return pl.pallas_call(
        _tc_body,
        out_shape=jax.ShapeDtypeStruct((G, C), jnp.float32),
    )(ng, amat, cntp, x, W1, b1.reshape(1, H), W2, b2.reshape(1, C))


# speedup vs baseline: 1.0893x; 1.0062x over previous
import jax
import jax.numpy as jnp
from jax import lax
from jax.experimental import pallas as pl
from jax.experimental.pallas import tpu as pltpu
from jax.experimental.pallas import tpu_sc as plsc

N = 10000
E = 320000
D = 128
H = 64
G = 64
C = 10

NS = 16
L = 16

NPAD = 10240
NSL = NPAD // NS
CNT_OFF = G * N
ASZ = G * N + 128
SL = ASZ // NS
SL2 = 8008
EC = E // NS
CB = 1024
RB = CB // 128
NB = (EC + CB - 1) // CB
ZB = 4016


def _invsqrt(v):
    i = lax.bitcast_convert_type(v, jnp.int32)
    i = jnp.int32(0x5F3759DF) - (i >> 1)
    y = lax.bitcast_convert_type(i, jnp.float32)
    for _ in range(3):
        y = y * (1.5 - 0.5 * v * y * y)
    return y


def _sc_body(src_hbm, dst_hbm, batch_hbm, out_hbm,
             batch_v, dinv_v, edge_v, degsl_v, idx0, val0, idx1, val1,
             idx2, val2, idx3, val3, onesf_v, onest_v,
             zeros_v, stage_v, sem0, sem1, sem2, sem3,
             a_sp, deg_sp, dinv_sp):
    s = lax.axis_index("s")
    iota = lax.iota(jnp.int32, L)
    zero16 = jnp.zeros((L,), jnp.float32)
    idxs = (idx0, idx1, idx2, idx3)
    vals = (val0, val1, val2, val3)
    sems = (sem0, sem1, sem2, sem3)

    din = [pltpu.async_copy(batch_hbm, batch_v, sem0),
           pltpu.async_copy(src_hbm.at[pl.ds(s * EC, EC)],
                            edge_v.at[pl.ds(0, EC)], sem1),
           pltpu.async_copy(dst_hbm.at[pl.ds(s * EC, EC)],
                            edge_v.at[pl.ds(EC, EC)], sem2)]

    def zloop(i, _):
        zeros_v[pl.ds(i * L, L)] = zero16
        return 0
    lax.fori_loop(0, ZB // L, zloop, 0)

    tail_valid = EC - (NB - 1) * CB

    def oloop(i, _):
        onesf_v[pl.ds(i * L, L)] = jnp.full((L,), 1.0, jnp.float32)
        onest_v[pl.ds(i * L, L)] = jnp.where(i * L + iota < tail_valid,
                                             1.0, 0.0)
        return 0
    lax.fori_loop(0, CB // L, oloop, 0)

    base = s * SL
    zd = [pltpu.async_copy(zeros_v.at[pl.ds(0, 4000)],
                           a_sp.at[pl.ds(base + j * 4000, 4000)], sem3)
          for j in range(9)]
    zd.append(pltpu.async_copy(zeros_v.at[pl.ds(0, SL - 36000)],
                               a_sp.at[pl.ds(base + 36000, SL - 36000)], sem3))
    pltpu.sync_copy(zeros_v.at[pl.ds(0, NSL)], deg_sp.at[pl.ds(s * NSL, NSL)])
    for d in din:
        d.wait()
    plsc.subcore_barrier()

    def fill_deg(idx_b, b):
        for k in range(CB // L):
            e0c = jnp.minimum(b * CB + k * L, EC - L)
            idx_b[pl.ds(k * L, L)] = edge_v[pl.ds(EC + e0c, L)]

    def fill(idx_b, val_b, b):
        for k in range(CB // L):
            e0 = b * CB + k * L
            e0c = jnp.minimum(e0, EC - L)
            ok = (e0 + iota) < EC
            col = k * L
            d16 = edge_v[pl.ds(EC + e0c, L)]
            s16 = edge_v[pl.ds(e0c, L)]
            dvs = plsc.load_gather(dinv_v, [s16])
            dvd = plsc.load_gather(dinv_v, [d16])
            g16 = plsc.load_gather(batch_v, [d16])
            idx_b[pl.ds(col, L)] = g16 * N + s16
            val_b[pl.ds(col, L)] = jnp.where(ok, dvs * dvd, 0.0)

    def p1(i, _):
        descs = []
        for q in range(4):
            fill_deg(idxs[q], 4 * i + q)
            descs.append(pltpu.async_copy(onesf_v, deg_sp.at[idxs[q]],
                                          sems[q], add=True))
        for d in descs:
            d.wait()
        return 0
    lax.fori_loop(0, NB // 4 - 1, p1, 0)
    descs = []
    for q in range(4):
        fill_deg(idxs[q], NB - 4 + q)
        vref = onest_v if q == 3 else onesf_v
        descs.append(pltpu.async_copy(vref, deg_sp.at[idxs[q]],
                                      sems[q], add=True))
    for d in descs:
        d.wait()
    plsc.subcore_barrier()

    pltpu.sync_copy(deg_sp.at[pl.ds(s * NSL, NSL)], degsl_v)

    def dloop(jj, _):
        dg = degsl_v[pl.ds(jj * L, L)] + 1.0
        dinv_v[pl.ds(s * NSL + jj * L, L)] = _invsqrt(dg)
        return 0
    lax.fori_loop(0, NSL // L, dloop, 0)
    pltpu.sync_copy(dinv_v.at[pl.ds(s * NSL, NSL)],
                    dinv_sp.at[pl.ds(s * NSL, NSL)])
    for d in zd:
        d.wait()
    plsc.subcore_barrier()
    pltpu.sync_copy(dinv_sp, dinv_v)

    def p2(i, _):
        descs = []
        for q in range(4):
            fill(idxs[q], vals[q], 4 * i + q)
            descs.append(pltpu.async_copy(vals[q], a_sp.at[idxs[q]],
                                          sems[q], add=True))
        for d in descs:
            d.wait()
        return 0
    lax.fori_loop(0, NB // 4, p2, 0)

    descs = []
    for half, (idx_b, val_b, sm) in enumerate(
            ((idx0, val0, sem0), (idx1, val1, sem1))):
        for jj in range(20):
            j = s * 40 + half * 20 + jj
            jc = jnp.minimum(j, N // L - 1)
            ok = (j * L + iota) < N
            i16 = jc * L + iota
            g16 = batch_v[pl.ds(jc * L, L)]
            dv = dinv_v[pl.ds(jc * L, L)]
            m, m2 = 2 * jj, 2 * jj + 1
            idx_b[pl.ds(m * L, L)] = g16 * N + i16
            val_b[pl.ds(m * L, L)] = jnp.where(ok, dv * dv, 0.0)
            idx_b[pl.ds(m2 * L, L)] = CNT_OFF + g16
            val_b[pl.ds(m2 * L, L)] = jnp.where(ok, 1.0, 0.0)
        for m in range(40, CB // L):
            val_b[pl.ds(m * L, L)] = zero16
        descs.append(pltpu.async_copy(val_b, a_sp.at[idx_b], sm, add=True))
    for d in descs:
        d.wait()
    plsc.subcore_barrier()

    outs = {}
    for j in range(11):
        buf = stage_v if j % 2 == 0 else zeros_v
        osem = sem1 if j % 2 == 0 else sem2
        sz = 4000 if j < 10 else 8
        off = s * SL + j * 4000
        if j >= 2:
            outs[j - 2].wait()
        pltpu.async_copy(a_sp.at[pl.ds(off, sz)], buf.at[pl.ds(0, sz)],
                         sem0).wait()
        outs[j] = pltpu.async_copy(buf.at[pl.ds(0, sz)],
                                   out_hbm.at[pl.ds(off, sz)], osem)
    outs[9].wait()
    outs[10].wait()


def _tc_body(ng_ref, a_ref, cnt_ref, x_ref, w1_ref, b1_ref, w2_ref, b2_ref,
             o_ref):
    p = jnp.dot(a_ref[...], x_ref[...], preferred_element_type=jnp.float32)
    cnt = cnt_ref[...]
    z = jnp.dot(p, w1_ref[...], preferred_element_type=jnp.float32)
    sums = z + cnt * b1_ref[...]
    valid = lax.broadcasted_iota(jnp.int32, (G, 1), 0) < ng_ref[0, 0]
    sums = jnp.where(valid, sums, 0.0)
    cntv = jnp.where(valid, cnt, 0.0)
    pooled = sums / jnp.maximum(cntv, 1.0)
    logits = jnp.dot(pooled, w2_ref[...],
                     preferred_element_type=jnp.float32) + b2_ref[...]
    mx = jnp.max(logits, axis=1, keepdims=True)
    lse = mx + jnp.log(jnp.sum(jnp.exp(logits - mx), axis=1, keepdims=True))
    o_ref[...] = logits - lse


def kernel(x, edge_index, batch, num_graphs, W1, b1, W2, b2):
    mesh = plsc.VectorSubcoreMesh(core_axis_name="c", subcore_axis_name="s",
                                  num_cores=1)
    sc = pl.kernel(
        _sc_body,
        out_type=jax.ShapeDtypeStruct((ASZ,), jnp.float32),
        mesh=mesh,
        compiler_params=pltpu.CompilerParams(needs_layout_passes=False),
        scratch_types=[
            pltpu.VMEM((N,), jnp.int32),
            pltpu.VMEM((NPAD,), jnp.float32),
            pltpu.VMEM((2 * EC,), jnp.int32),
            pltpu.VMEM((NSL,), jnp.float32),
            pltpu.VMEM((CB,), jnp.int32),
            pltpu.VMEM((CB,), jnp.float32),
            pltpu.VMEM((CB,), jnp.int32),
            pltpu.VMEM((CB,), jnp.float32),
            pltpu.VMEM((CB,), jnp.int32),
            pltpu.VMEM((CB,), jnp.float32),
            pltpu.VMEM((CB,), jnp.int32),
            pltpu.VMEM((CB,), jnp.float32),
            pltpu.VMEM((CB,), jnp.float32),
            pltpu.VMEM((CB,), jnp.float32),
            pltpu.VMEM((ZB,), jnp.float32),
            pltpu.VMEM((SL2,), jnp.float32),
            pltpu.SemaphoreType.DMA,
            pltpu.SemaphoreType.DMA,
            pltpu.SemaphoreType.DMA,
            pltpu.SemaphoreType.DMA,
            pltpu.VMEM_SHARED((ASZ,), jnp.float32),
            pltpu.VMEM_SHARED((NPAD,), jnp.float32),
            pltpu.VMEM_SHARED((NPAD,), jnp.float32),
        ],
    )
    a2 = sc(edge_index[0], edge_index[1], batch)
    amat = a2[:G * N].reshape(G, N)
    cntp = a2[CNT_OFF:CNT_OFF + G].reshape(G, 1)
    ng = jnp.asarray(num_graphs, jnp.int32).reshape(1, 1)
    return pl.pallas_call(
        _tc_body,
        out_shape=jax.ShapeDtypeStruct((G, C), jnp.float32),
    )(ng, amat, cntp, x, W1, b1.reshape(1, H), W2, b2.reshape(1, C))
